# baseline (speedup 1.0000x reference)
import functools

import jax
import jax.numpy as jnp
from jax import lax
from jax.experimental import pallas as pl
from jax.experimental.pallas import tpu as pltpu
from jax.experimental.pallas import tpu_sc as plsc

N_LABELS = 1000
PAD_LABELS = 1024
EMBED = 64
BATCH = 4096
T_STEPS = 200
T_OUT = T_STEPS - 1
T_PAD = 256
NUM_CORES = 2
NUM_SUBCORES = 16
NW = NUM_CORES * NUM_SUBCORES
ROWS_W = BATCH // NW
IDX_W = ROWS_W * T_PAD
N_GROUP = 4
IDX_G = IDX_W // N_GROUP
ROW_BLOCK = 128
SEQ_BLOCK = 512

NEG = -1e30


def _tc_idx_body(seq_ref, o_ref):
    sq = seq_ref[...]
    sqp = jnp.pad(sq, ((0, 0), (0, T_PAD + 1 - T_STEPS)))
    src = sqp[:, :T_PAD]
    tgt = sqp[:, 1:T_PAD + 1]
    o_ref[...] = ((src << 10) | tgt).reshape(SEQ_BLOCK * T_PAD)


def _tc_idx(label_sequences):
    return pl.pallas_call(
        _tc_idx_body,
        grid=(BATCH // SEQ_BLOCK,),
        in_specs=[pl.BlockSpec((SEQ_BLOCK, T_STEPS), lambda i: (i, 0))],
        out_specs=pl.BlockSpec((SEQ_BLOCK * T_PAD,), lambda i: (i,)),
        out_shape=jax.ShapeDtypeStruct((BATCH * T_PAD,), jnp.int32),
    )(label_sequences)


def _tc_table_body(s_ref, t_ref, o_ref):
    s = s_ref[...]
    t = t_ref[...]
    logits = lax.dot_general(
        s, t, (((1,), (1,)), ((), ())),
        preferred_element_type=jnp.float32,
        precision=lax.Precision.HIGHEST,
    )
    col = lax.broadcasted_iota(jnp.int32, logits.shape, 1)
    logits = jnp.where(col < N_LABELS, logits, NEG)
    m = jnp.max(logits, axis=1, keepdims=True)
    lse = jnp.log(jnp.sum(jnp.exp(logits - m), axis=1, keepdims=True)) + m
    o_ref[...] = (logits - lse).reshape(ROW_BLOCK * PAD_LABELS)


def _tc_table(s_pad, t_pad):
    return pl.pallas_call(
        _tc_table_body,
        grid=(PAD_LABELS // ROW_BLOCK,),
        in_specs=[
            pl.BlockSpec((ROW_BLOCK, EMBED), lambda i: (i, 0)),
            pl.BlockSpec((PAD_LABELS, EMBED), lambda i: (0, 0)),
        ],
        out_specs=pl.BlockSpec((ROW_BLOCK * PAD_LABELS,), lambda i: (i,)),
        out_shape=jax.ShapeDtypeStruct((PAD_LABELS * PAD_LABELS,), jnp.float32),
    )(s_pad, t_pad)


@functools.partial(
    pl.kernel,
    out_type=jax.ShapeDtypeStruct((BATCH * T_PAD,), jnp.float32),
    mesh=plsc.VectorSubcoreMesh(
        core_axis_name="c", subcore_axis_name="s"),
    scratch_types=[
        pltpu.VMEM((IDX_W,), jnp.int32),
        pltpu.VMEM((IDX_W,), jnp.float32),
        pltpu.SemaphoreType.DMA,
    ],
)
def _sc_gather(lp_hbm, idx_hbm, out_hbm, idx_v, val_v, sem):
    wid = lax.axis_index("s") * NUM_CORES + lax.axis_index("c")
    base = wid * IDX_W
    for g in range(N_GROUP):
        pltpu.sync_copy(idx_hbm.at[pl.ds(base + g * IDX_G, IDX_G)],
                        idx_v.at[pl.ds(g * IDX_G, IDX_G)])
        pltpu.async_copy(
            lp_hbm.at[idx_v.at[pl.ds(g * IDX_G, IDX_G)]],
            val_v.at[pl.ds(g * IDX_G, IDX_G)],
            sem,
        )
    pltpu.make_async_copy(
        lp_hbm.at[pl.ds(0, IDX_W)],
        val_v,
        sem,
    ).wait()
    pltpu.sync_copy(val_v, out_hbm.at[pl.ds(base, IDX_W)])


def kernel(label_sequences, source_embeddings, target_embeddings):
    seq = label_sequences.astype(jnp.int32)
    s_pad = jnp.pad(source_embeddings, ((0, PAD_LABELS - N_LABELS), (0, 0)))
    t_pad = jnp.pad(target_embeddings, ((0, PAD_LABELS - N_LABELS), (0, 0)))
    idx = _tc_idx(seq)
    table = _tc_table(s_pad, t_pad)
    flat = _sc_gather(table, idx)
    return flat.reshape(BATCH, T_PAD)[:, :T_OUT]


# speedup vs baseline: 12.3577x; 12.3577x over previous
import functools

import jax
import jax.numpy as jnp
from jax import lax
from jax.experimental import pallas as pl
from jax.experimental.pallas import tpu as pltpu
from jax.experimental.pallas import tpu_sc as plsc

N_LABELS = 1000
PAD_LABELS = 1024
EMBED = 64
BATCH = 4096
T_STEPS = 200
T_OUT = T_STEPS - 1
T_PAD = 256
NUM_CORES = 2
NUM_SUBCORES = 16
NW = NUM_CORES * NUM_SUBCORES
ROWS_W = BATCH // NW
IDX_W = ROWS_W * T_PAD
N_GROUP = 4
IDX_G = IDX_W // N_GROUP
ROW_BLOCK = 128
SEQ_BLOCK = 512

NEG = -1e30


def _tc_idx_body(seq_ref, o_ref):
    sq = seq_ref[...]
    sqp = jnp.pad(sq, ((0, 0), (0, T_PAD + 1 - T_STEPS)))
    src = sqp[:, :T_PAD]
    tgt = sqp[:, 1:T_PAD + 1]
    pair = (src << 10) | tgt
    i = pl.program_id(0)
    row = lax.broadcasted_iota(jnp.int32, pair.shape, 0)
    col = lax.broadcasted_iota(jnp.int32, pair.shape, 1)
    q = (i * SEQ_BLOCK + row) * T_PAD + col
    idx = jnp.where(col < T_STEPS, pair, q)
    o_ref[...] = idx.reshape(SEQ_BLOCK * T_PAD)


def _tc_idx(label_sequences):
    return pl.pallas_call(
        _tc_idx_body,
        grid=(BATCH // SEQ_BLOCK,),
        in_specs=[pl.BlockSpec((SEQ_BLOCK, T_STEPS), lambda i: (i, 0))],
        out_specs=pl.BlockSpec((SEQ_BLOCK * T_PAD,), lambda i: (i,)),
        out_shape=jax.ShapeDtypeStruct((BATCH * T_PAD,), jnp.int32),
    )(label_sequences)


def _tc_table_body(s_ref, t_ref, o_ref):
    s = s_ref[...]
    t = t_ref[...]
    logits = lax.dot_general(
        s, t, (((1,), (1,)), ((), ())),
        preferred_element_type=jnp.float32,
        precision=lax.Precision.HIGHEST,
    )
    col = lax.broadcasted_iota(jnp.int32, logits.shape, 1)
    logits = jnp.where(col < N_LABELS, logits, NEG)
    m = jnp.max(logits, axis=1, keepdims=True)
    lse = jnp.log(jnp.sum(jnp.exp(logits - m), axis=1, keepdims=True)) + m
    o_ref[...] = (logits - lse).reshape(ROW_BLOCK * PAD_LABELS)


def _tc_table(s_pad, t_pad):
    return pl.pallas_call(
        _tc_table_body,
        grid=(PAD_LABELS // ROW_BLOCK,),
        in_specs=[
            pl.BlockSpec((ROW_BLOCK, EMBED), lambda i: (i, 0)),
            pl.BlockSpec((PAD_LABELS, EMBED), lambda i: (0, 0)),
        ],
        out_specs=pl.BlockSpec((ROW_BLOCK * PAD_LABELS,), lambda i: (i,)),
        out_shape=jax.ShapeDtypeStruct((PAD_LABELS * PAD_LABELS,), jnp.float32),
    )(s_pad, t_pad)


@functools.partial(
    pl.kernel,
    out_type=jax.ShapeDtypeStruct((BATCH * T_PAD,), jnp.float32),
    mesh=plsc.VectorSubcoreMesh(
        core_axis_name="c", subcore_axis_name="s"),
    scratch_types=[
        pltpu.VMEM((IDX_W,), jnp.int32),
        pltpu.VMEM((IDX_W,), jnp.float32),
        pltpu.SemaphoreType.DMA,
    ],
)
def _sc_gather(lp_hbm, idx_hbm, out_hbm, idx_v, val_v, sem):
    wid = lax.axis_index("s") * NUM_CORES + lax.axis_index("c")
    base = wid * IDX_W
    for g in range(N_GROUP):
        pltpu.sync_copy(idx_hbm.at[pl.ds(base + g * IDX_G, IDX_G)],
                        idx_v.at[pl.ds(g * IDX_G, IDX_G)])
        pltpu.async_copy(
            lp_hbm.at[idx_v.at[pl.ds(g * IDX_G, IDX_G)]],
            val_v.at[pl.ds(g * IDX_G, IDX_G)],
            sem,
        )
    pltpu.make_async_copy(
        lp_hbm.at[pl.ds(0, IDX_W)],
        val_v,
        sem,
    ).wait()
    pltpu.sync_copy(val_v, out_hbm.at[pl.ds(base, IDX_W)])


def kernel(label_sequences, source_embeddings, target_embeddings):
    seq = label_sequences.astype(jnp.int32)
    s_pad = jnp.pad(source_embeddings, ((0, PAD_LABELS - N_LABELS), (0, 0)))
    t_pad = jnp.pad(target_embeddings, ((0, PAD_LABELS - N_LABELS), (0, 0)))
    idx = _tc_idx(seq)
    table = _tc_table(s_pad, t_pad)
    flat = _sc_gather(table, idx)
    return flat.reshape(BATCH, T_PAD)[:, :T_OUT]


# speedup vs baseline: 13.9038x; 1.1251x over previous
import functools

import jax
import jax.numpy as jnp
from jax import lax
from jax.experimental import pallas as pl
from jax.experimental.pallas import tpu as pltpu
from jax.experimental.pallas import tpu_sc as plsc

N_LABELS = 1000
PAD_LABELS = 1024
EMBED = 64
BATCH = 4096
T_STEPS = 200
T_OUT = T_STEPS - 1
NUM_CORES = 2
NUM_SUBCORES = 16
NW = NUM_CORES * NUM_SUBCORES
N_PAIRS = BATCH * T_OUT
IDX_W = N_PAIRS // NW
N_GROUP = 4
IDX_G = IDX_W // N_GROUP
ROW_BLOCK = 128
SEQ_BLOCK = 1024
N_SEQ_BLOCKS = BATCH // SEQ_BLOCK

NEG = -1e30


def _tc_idx_body(seq_ref, o_ref):
    sq = seq_ref[...]
    src = sq[:, :T_OUT]
    tgt = sq[:, 1:T_STEPS]
    pair = (src << 10) | tgt
    o_ref[...] = pair.T.reshape(SEQ_BLOCK * T_OUT)


def _tc_idx(label_sequences):
    return pl.pallas_call(
        _tc_idx_body,
        grid=(N_SEQ_BLOCKS,),
        in_specs=[pl.BlockSpec((SEQ_BLOCK, T_STEPS), lambda i: (i, 0))],
        out_specs=pl.BlockSpec((SEQ_BLOCK * T_OUT,), lambda i: (i,)),
        out_shape=jax.ShapeDtypeStruct((N_PAIRS,), jnp.int32),
    )(label_sequences)


def _tc_table_body(s_ref, t_ref, o_ref):
    s = s_ref[...]
    t = t_ref[...]
    logits = lax.dot_general(
        s, t, (((1,), (1,)), ((), ())),
        preferred_element_type=jnp.float32,
        precision=lax.Precision.HIGHEST,
    )
    col = lax.broadcasted_iota(jnp.int32, logits.shape, 1)
    logits = jnp.where(col < N_LABELS, logits, NEG)
    m = jnp.max(logits, axis=1, keepdims=True)
    lse = jnp.log(jnp.sum(jnp.exp(logits - m), axis=1, keepdims=True)) + m
    o_ref[...] = (logits - lse).reshape(ROW_BLOCK * PAD_LABELS)


def _tc_table(s_pad, t_pad):
    return pl.pallas_call(
        _tc_table_body,
        grid=(PAD_LABELS // ROW_BLOCK,),
        in_specs=[
            pl.BlockSpec((ROW_BLOCK, EMBED), lambda i: (i, 0)),
            pl.BlockSpec((PAD_LABELS, EMBED), lambda i: (0, 0)),
        ],
        out_specs=pl.BlockSpec((ROW_BLOCK * PAD_LABELS,), lambda i: (i,)),
        out_shape=jax.ShapeDtypeStruct((PAD_LABELS * PAD_LABELS,), jnp.float32),
    )(s_pad, t_pad)


@functools.partial(
    pl.kernel,
    out_type=jax.ShapeDtypeStruct((N_PAIRS,), jnp.float32),
    mesh=plsc.VectorSubcoreMesh(
        core_axis_name="c", subcore_axis_name="s"),
    scratch_types=[
        pltpu.VMEM((IDX_W,), jnp.int32),
        pltpu.VMEM((IDX_W,), jnp.float32),
        pltpu.SemaphoreType.DMA,
    ],
)
def _sc_gather(lp_hbm, idx_hbm, out_hbm, idx_v, val_v, sem):
    wid = lax.axis_index("s") * NUM_CORES + lax.axis_index("c")
    base = wid * IDX_W
    for g in range(N_GROUP):
        pltpu.sync_copy(idx_hbm.at[pl.ds(base + g * IDX_G, IDX_G)],
                        idx_v.at[pl.ds(g * IDX_G, IDX_G)])
        pltpu.async_copy(
            lp_hbm.at[idx_v.at[pl.ds(g * IDX_G, IDX_G)]],
            val_v.at[pl.ds(g * IDX_G, IDX_G)],
            sem,
        )
    pltpu.make_async_copy(
        lp_hbm.at[pl.ds(0, IDX_W)],
        val_v,
        sem,
    ).wait()
    pltpu.sync_copy(val_v, out_hbm.at[pl.ds(base, IDX_W)])


def kernel(label_sequences, source_embeddings, target_embeddings):
    s_pad = jnp.pad(source_embeddings, ((0, PAD_LABELS - N_LABELS), (0, 0)))
    t_pad = jnp.pad(target_embeddings, ((0, PAD_LABELS - N_LABELS), (0, 0)))
    idx = _tc_idx(label_sequences.astype(jnp.int32))
    table = _tc_table(s_pad, t_pad)
    flat = _sc_gather(table, idx)
    return (flat.reshape(N_SEQ_BLOCKS, T_OUT, SEQ_BLOCK)
            .transpose(0, 2, 1)
            .reshape(BATCH, T_OUT))


# speedup vs baseline: 14.1640x; 1.0187x over previous
import functools

import jax
import jax.numpy as jnp
from jax import lax
from jax.experimental import pallas as pl
from jax.experimental.pallas import tpu as pltpu
from jax.experimental.pallas import tpu_sc as plsc

N_LABELS = 1000
PAD_LABELS = 1024
EMBED = 64
BATCH = 4096
T_STEPS = 200
T_OUT = T_STEPS - 1
NUM_CORES = 2
NUM_SUBCORES = 16
NW = NUM_CORES * NUM_SUBCORES
N_PAIRS = BATCH * T_OUT
IDX_W = N_PAIRS // NW
N_GROUP = 8
IDX_G = IDX_W // N_GROUP
ROW_BLOCK = 256
SEQ_BLOCK = 1024
N_SEQ_BLOCKS = BATCH // SEQ_BLOCK

NEG = -1e30


def _tc_idx_body(seq_ref, o_ref):
    sq = seq_ref[...]
    src = sq[:, :T_OUT]
    tgt = sq[:, 1:T_STEPS]
    pair = (src << 10) | tgt
    o_ref[...] = pair.T.reshape(SEQ_BLOCK * T_OUT)


def _tc_idx(label_sequences):
    return pl.pallas_call(
        _tc_idx_body,
        grid=(N_SEQ_BLOCKS,),
        in_specs=[pl.BlockSpec((SEQ_BLOCK, T_STEPS), lambda i: (i, 0))],
        out_specs=pl.BlockSpec((SEQ_BLOCK * T_OUT,), lambda i: (i,)),
        out_shape=jax.ShapeDtypeStruct((N_PAIRS,), jnp.int32),
    )(label_sequences)


def _tc_table_body(s_ref, t_ref, o_ref):
    s = s_ref[...]
    t = t_ref[...]
    logits = lax.dot_general(
        s, t, (((1,), (1,)), ((), ())),
        preferred_element_type=jnp.float32,
        precision=lax.Precision.HIGHEST,
    )
    col = lax.broadcasted_iota(jnp.int32, logits.shape, 1)
    logits = jnp.where(col < N_LABELS, logits, NEG)
    m = jnp.max(logits, axis=1, keepdims=True)
    lse = jnp.log(jnp.sum(jnp.exp(logits - m), axis=1, keepdims=True)) + m
    o_ref[...] = (logits - lse).reshape(ROW_BLOCK * PAD_LABELS)


def _tc_table(s_pad, t_pad):
    return pl.pallas_call(
        _tc_table_body,
        grid=(PAD_LABELS // ROW_BLOCK,),
        in_specs=[
            pl.BlockSpec((ROW_BLOCK, EMBED), lambda i: (i, 0)),
            pl.BlockSpec((PAD_LABELS, EMBED), lambda i: (0, 0)),
        ],
        out_specs=pl.BlockSpec((ROW_BLOCK * PAD_LABELS,), lambda i: (i,)),
        out_shape=jax.ShapeDtypeStruct((PAD_LABELS * PAD_LABELS,), jnp.float32),
    )(s_pad, t_pad)


@functools.partial(
    pl.kernel,
    out_type=jax.ShapeDtypeStruct((N_PAIRS,), jnp.float32),
    mesh=plsc.VectorSubcoreMesh(
        core_axis_name="c", subcore_axis_name="s"),
    scratch_types=[
        pltpu.VMEM((IDX_W,), jnp.int32),
        pltpu.VMEM((IDX_W,), jnp.float32),
        pltpu.SemaphoreType.DMA,
    ],
)
def _sc_gather(lp_hbm, idx_hbm, out_hbm, idx_v, val_v, sem):
    wid = lax.axis_index("s") * NUM_CORES + lax.axis_index("c")
    base = wid * IDX_W
    for g in range(N_GROUP):
        pltpu.sync_copy(idx_hbm.at[pl.ds(base + g * IDX_G, IDX_G)],
                        idx_v.at[pl.ds(g * IDX_G, IDX_G)])
        pltpu.async_copy(
            lp_hbm.at[idx_v.at[pl.ds(g * IDX_G, IDX_G)]],
            val_v.at[pl.ds(g * IDX_G, IDX_G)],
            sem,
        )
    pltpu.make_async_copy(
        lp_hbm.at[pl.ds(0, IDX_W)],
        val_v,
        sem,
    ).wait()
    pltpu.sync_copy(val_v, out_hbm.at[pl.ds(base, IDX_W)])


def kernel(label_sequences, source_embeddings, target_embeddings):
    s_pad = jnp.pad(source_embeddings, ((0, PAD_LABELS - N_LABELS), (0, 0)))
    t_pad = jnp.pad(target_embeddings, ((0, PAD_LABELS - N_LABELS), (0, 0)))
    idx = _tc_idx(label_sequences)
    table = _tc_table(s_pad, t_pad)
    flat = _sc_gather(table, idx)
    return (flat.reshape(N_SEQ_BLOCKS, T_OUT, SEQ_BLOCK)
            .transpose(0, 2, 1)
            .reshape(BATCH, T_OUT))
